# baseline (speedup 1.0000x reference)
import functools

import jax
import jax.numpy as jnp
from jax import lax
from jax.experimental import pallas as pl
from jax.experimental.pallas import tpu as pltpu
from jax.experimental.pallas import tpu_sc as plsc

HIDDEN = 64
BATCH = 16384
HIST = 50
NC, NS = 2, 16
NW = NC * NS
ROWS_W = BATCH // NW
RCHUNK = 16
N_CHUNKS = ROWS_W // RCHUNK
N_PAIRS = N_CHUNKS // 2

_mesh = plsc.VectorSubcoreMesh(core_axis_name="c", subcore_axis_name="s")


@functools.partial(
    pl.kernel,
    mesh=_mesh,
    out_type=jax.ShapeDtypeStruct((BATCH, HIST, HIDDEN), jnp.float32),
    scratch_types=[
        pltpu.VMEM((ROWS_W, HIST), jnp.int32),
        pltpu.VMEM((RCHUNK, HIST, HIDDEN), jnp.float32),
        pltpu.VMEM((RCHUNK, HIST, HIDDEN), jnp.float32),
        pltpu.SemaphoreType.DMA,
        pltpu.SemaphoreType.DMA,
        pltpu.SemaphoreType.DMA,
        pltpu.SemaphoreType.DMA,
    ],
    compiler_params=pltpu.CompilerParams(use_tc_tiling_on_sc=False),
)
def _gather(idx_hbm, table_hbm, out_hbm, idx_v, rb0, rb1, g0, g1, s0, s1):
    wid = lax.axis_index("s") * NC + lax.axis_index("c")
    base = wid * ROWS_W

    pltpu.sync_copy(idx_hbm.at[pl.ds(base, ROWS_W)], idx_v)

    class _GChunk:
        def __init__(self, c, rbuf, sem):
            self.c, self.rbuf, self.sem = c, rbuf, sem

        def _one(self, j):
            return pltpu.make_async_copy(
                table_hbm.at[idx_v.at[self.c * RCHUNK + j]],
                self.rbuf.at[j], self.sem)

        def start(self):
            for j in range(RCHUNK):
                self._one(j).start()

        def wait(self):
            for j in range(RCHUNK):
                self._one(j).wait()

    def g_copy(c, rbuf, sem):
        return _GChunk(c, rbuf, sem)

    def s_copy(c, rbuf, sem):
        return pltpu.make_async_copy(
            rbuf, out_hbm.at[pl.ds(base + c * RCHUNK, RCHUNK)], sem)

    g_copy(0, rb0, g0).start()
    g_copy(0, rb0, g0).wait()
    g_copy(1, rb1, g1).start()
    s_copy(0, rb0, s0).start()

    def body(p, carry):
        c0 = 2 * p
        g_copy(c0 - 1, rb1, g1).wait()
        s_copy(c0 - 2, rb0, s0).wait()
        g_copy(c0, rb0, g0).start()
        s_copy(c0 - 1, rb1, s1).start()
        g_copy(c0, rb0, g0).wait()
        s_copy(c0 - 1, rb1, s1).wait()
        g_copy(c0 + 1, rb1, g1).start()
        s_copy(c0, rb0, s0).start()
        return carry

    lax.fori_loop(1, N_PAIRS, body, 0)

    g_copy(N_CHUNKS - 1, rb1, g1).wait()
    s_copy(N_CHUNKS - 1, rb1, s1).start()
    s_copy(N_CHUNKS - 2, rb0, s0).wait()
    s_copy(N_CHUNKS - 1, rb1, s1).wait()


def kernel(input_ids, embedding):
    return _gather(input_ids, embedding)
